# baseline (speedup 1.0000x reference)
import functools
import math

import jax
import jax.numpy as jnp
from jax import lax
from jax.experimental import pallas as pl
from jax.experimental.pallas import tpu as pltpu
from jax.experimental.pallas import tpu_sc as plsc

VOCAB = 1000000
D = 64
BATCH = 4096
SEQ = 200
B = BATCH * SEQ
SCALE = math.sqrt(D)

NC = 2
NS = 16
NW = NC * NS
B_PER_W = B // NW

IDX_MINOR = 128
CHUNK = 512
G_PER_CHUNK = CHUNK // IDX_MINOR
CHUNKS = B_PER_W // CHUNK
IDX_ROWS_PER_CHUNK = CHUNK // IDX_MINOR


def _embed_kernel(x_hbm, table_hbm, out_hbm, idx_v, rows_v, gsem):
    wid = lax.axis_index("s") * NC + lax.axis_index("c")
    idx_row0 = wid * (B_PER_W // IDX_MINOR)
    out_row0 = wid * B_PER_W

    def chunk_body(i, _):
        pltpu.sync_copy(
            x_hbm.at[pl.ds(idx_row0 + i * IDX_ROWS_PER_CHUNK, IDX_ROWS_PER_CHUNK)],
            idx_v,
        )
        copies = []
        for j in range(G_PER_CHUNK):
            copies.append(
                pltpu.async_copy(
                    table_hbm.at[idx_v.at[j]],
                    rows_v.at[pl.ds(j * IDX_MINOR, IDX_MINOR)],
                    gsem,
                )
            )
        for c in copies:
            c.wait()

        def scale_row(r, _):
            for j in range(D // 16):
                sl = pl.ds(j * 16, 16)
                rows_v[r, sl] = rows_v[r, sl] * SCALE
            return 0

        lax.fori_loop(0, CHUNK, scale_row, 0)

        pltpu.sync_copy(rows_v, out_hbm.at[pl.ds(out_row0 + i * CHUNK, CHUNK)])
        return 0

    lax.fori_loop(0, CHUNKS, chunk_body, 0)


@jax.jit
def kernel(x, table):
    x2 = x.reshape(B // IDX_MINOR, IDX_MINOR).astype(jnp.int32)
    mesh = plsc.VectorSubcoreMesh(
        core_axis_name="c", subcore_axis_name="s", num_cores=NC, num_subcores=NS
    )
    out = pl.kernel(
        _embed_kernel,
        out_type=jax.ShapeDtypeStruct((B, D), jnp.float32),
        mesh=mesh,
        scratch_types=[
            pltpu.VMEM((IDX_ROWS_PER_CHUNK, IDX_MINOR), jnp.int32),
            pltpu.VMEM((CHUNK, D), jnp.float32),
            pltpu.SemaphoreType.DMA,
        ],
        compiler_params=pltpu.CompilerParams(use_tc_tiling_on_sc=False),
    )(x2, table)
    return out.reshape(BATCH, SEQ, D)


# speedup vs baseline: 1.1358x; 1.1358x over previous
import math

import jax
import jax.numpy as jnp
from jax import lax
from jax.experimental import pallas as pl
from jax.experimental.pallas import tpu as pltpu
from jax.experimental.pallas import tpu_sc as plsc

VOCAB = 1000000
D = 64
BATCH = 4096
SEQ = 200
B = BATCH * SEQ
SCALE = math.sqrt(D)

NC = 2
NS = 16
NW = NC * NS
B_PER_W = B // NW

IDX_MINOR = 128
IDX_ROWS_PER_W = B_PER_W // IDX_MINOR
CHUNK = 512
G_PER_CHUNK = CHUNK // IDX_MINOR
CHUNKS = B_PER_W // CHUNK


def _embed_kernel(x_hbm, table_hbm, out_hbm,
                  idx_v, rows_v0, rows_v1, gsem0, gsem1, osem0, osem1):
    wid = lax.axis_index("s") * NC + lax.axis_index("c")
    out_row0 = wid * B_PER_W

    pltpu.sync_copy(x_hbm.at[pl.ds(wid * IDX_ROWS_PER_W, IDX_ROWS_PER_W)], idx_v)

    def fire_gathers(i, rows_vb, gsemb):
        for j in range(G_PER_CHUNK):
            pltpu.async_copy(
                table_hbm.at[idx_v.at[i * G_PER_CHUNK + j]],
                rows_vb.at[pl.ds(j * IDX_MINOR, IDX_MINOR)],
                gsemb,
            )

    def wait_gathers(rows_vb, gsemb):
        pltpu.make_async_copy(table_hbm.at[pl.ds(0, CHUNK)], rows_vb, gsemb).wait()

    def scale_chunk(rows_vb):
        def srow(r, _):
            for j in range(D // 16):
                sl = pl.ds(j * 16, 16)
                rows_vb[r, sl] = rows_vb[r, sl] * SCALE
            return 0
        lax.fori_loop(0, CHUNK, srow, 0, unroll=4)

    def fire_store(i, rows_vb, osemb):
        pltpu.async_copy(rows_vb, out_hbm.at[pl.ds(out_row0 + i * CHUNK, CHUNK)],
                         osemb)

    def wait_store(rows_vb, osemb):
        pltpu.make_async_copy(rows_vb, out_hbm.at[pl.ds(0, CHUNK)], osemb).wait()

    fire_gathers(0, rows_v0, gsem0)

    def pair_body(g, _):
        for b in range(2):
            i = 2 * g + b
            if b == 0:
                cur_rows, cur_g, cur_o = rows_v0, gsem0, osem0
                nxt_rows, nxt_g, nxt_o = rows_v1, gsem1, osem1
            else:
                cur_rows, cur_g, cur_o = rows_v1, gsem1, osem1
                nxt_rows, nxt_g, nxt_o = rows_v0, gsem0, osem0

            @pl.when(i + 1 < CHUNKS)
            def _():
                @pl.when(i >= 1)
                def _():
                    wait_store(nxt_rows, nxt_o)
                fire_gathers(i + 1, nxt_rows, nxt_g)

            wait_gathers(cur_rows, cur_g)
            scale_chunk(cur_rows)
            fire_store(i, cur_rows, cur_o)
        return 0

    lax.fori_loop(0, CHUNKS // 2, pair_body, 0)
    wait_store(rows_v0, osem0)
    wait_store(rows_v1, osem1)


@jax.jit
def kernel(x, table):
    x2 = x.reshape(B // IDX_MINOR, IDX_MINOR).astype(jnp.int32)
    mesh = plsc.VectorSubcoreMesh(
        core_axis_name="c", subcore_axis_name="s", num_cores=NC, num_subcores=NS
    )
    out = pl.kernel(
        _embed_kernel,
        out_type=jax.ShapeDtypeStruct((B, D), jnp.float32),
        mesh=mesh,
        scratch_types=[
            pltpu.VMEM((IDX_ROWS_PER_W, IDX_MINOR), jnp.int32),
            pltpu.VMEM((CHUNK, D), jnp.float32),
            pltpu.VMEM((CHUNK, D), jnp.float32),
            pltpu.SemaphoreType.DMA,
            pltpu.SemaphoreType.DMA,
            pltpu.SemaphoreType.DMA,
            pltpu.SemaphoreType.DMA,
        ],
        compiler_params=pltpu.CompilerParams(use_tc_tiling_on_sc=False),
    )(x2, table)
    return out.reshape(BATCH, SEQ, D)
